# baseline (speedup 1.0000x reference)
import functools

import jax
import jax.numpy as jnp
from jax import lax
from jax.experimental import pallas as pl
from jax.experimental.pallas import tpu as pltpu
from jax.experimental.pallas import tpu_sc as plsc


def _make_gather(D, B, T):
    info = plsc.get_sparse_core_info()
    NC, NS, L = 1, info.num_subcores, info.num_lanes
    NW = NC * NS
    assert B % NW == 0 and (B // NW) % L == 0
    b_per_w = B // NW
    mesh = plsc.VectorSubcoreMesh(
        core_axis_name="c", subcore_axis_name="s", num_cores=NC, num_subcores=NS
    )

    @functools.partial(
        pl.kernel,
        mesh=mesh,
        out_type=jax.ShapeDtypeStruct((B, D), jnp.float32),
        scratch_types=[
            pltpu.VMEM((b_per_w,), jnp.int32),
            pltpu.VMEM((b_per_w, D), jnp.float32),
            pltpu.SemaphoreType.DMA,
        ],
    )
    def gather(table_hbm, ts_hbm, out_hbm, idx_v, rows_v, sem):
        wid = lax.axis_index("s") * NC + lax.axis_index("c")
        base = wid * b_per_w
        pltpu.sync_copy(ts_hbm.at[pl.ds(base, b_per_w)], idx_v)
        for g in range(b_per_w // L):
            ts = idx_v[pl.ds(g * L, L)]
            rows = base + g * L + lax.iota(jnp.int32, L)
            idx_v[pl.ds(g * L, L)] = rows * T + ts
        pltpu.async_copy(table_hbm.at[idx_v], rows_v, sem).wait()
        pltpu.sync_copy(rows_v, out_hbm.at[pl.ds(base, b_per_w)])

    return gather


def _mm_partial_kernel(el_ref, er_ref, w_ref, b_ref, part_ref):
    D = el_ref.shape[1]
    part_ref[...] = (
        jnp.dot(el_ref[...], w_ref[0:D, :], preferred_element_type=jnp.float32)
        + jnp.dot(er_ref[...], w_ref[D : 2 * D, :], preferred_element_type=jnp.float32)
        + b_ref[...]
    )


def _mm_final_kernel(part_ref, et_ref, w_ref, h_ref):
    D = et_ref.shape[1]
    h_ref[...] = part_ref[...] + jnp.dot(
        et_ref[...], w_ref[2 * D : 3 * D, :], preferred_element_type=jnp.float32
    )


def kernel(inf_enc_seq, inf_enc_key_seq, e_l, e_r, start_ind, end_ind, timestep, W, b):
    B, T, D = inf_enc_seq.shape
    NZ = W.shape[1] // 2
    table = inf_enc_seq.reshape(B * T, D)
    ts = timestep.reshape(B).astype(jnp.int32)
    e_tilde = _make_gather(D, B, T)(table, ts)
    part = pl.pallas_call(
        _mm_partial_kernel,
        out_shape=jax.ShapeDtypeStruct((B, 2 * NZ), jnp.float32),
    )(e_l, e_r, W, b.reshape(1, 2 * NZ))
    h = pl.pallas_call(
        _mm_final_kernel,
        out_shape=jax.ShapeDtypeStruct((B, 2 * NZ), jnp.float32),
    )(part, e_tilde, W)
    return (h[:, :NZ], h[:, NZ:])
